# Initial kernel scaffold; baseline (speedup 1.0000x reference)
import jax
import jax.numpy as jnp
from jax.experimental import pallas as pl


def kernel(x, rel_pos_emb):
    raise NotImplementedError("write your pallas kernel here")


# speedup vs baseline: 2.5806x; 2.5806x over previous
import functools

import jax
import jax.numpy as jnp
from jax import lax
from jax.experimental import pallas as pl
from jax.experimental.pallas import tpu as pltpu
from jax.experimental.pallas import tpu_sc as plsc

_MAX_LEN = 8000
_NUM_WORKERS = 32
_CHUNK = 32768


def _make_sc_copy(n_elems: int, elem_start: int):
    per_w = n_elems // _NUM_WORKERS
    assert per_w * _NUM_WORKERS == n_elems and per_w % 8 == 0

    sizes = []
    off = 0
    while off < per_w:
        sizes.append(min(_CHUNK, per_w - off))
        off += sizes[-1]
    offs = [sum(sizes[:i]) for i in range(len(sizes))]
    n_chunks = len(sizes)

    mesh = plsc.VectorSubcoreMesh(core_axis_name="c", subcore_axis_name="s")

    @functools.partial(
        pl.kernel,
        out_type=jax.ShapeDtypeStruct((n_elems,), jnp.float32),
        mesh=mesh,
        scratch_types=[
            pltpu.VMEM((_CHUNK,), jnp.float32),
            pltpu.VMEM((_CHUNK,), jnp.float32),
            pltpu.SemaphoreType.DMA,
            pltpu.SemaphoreType.DMA,
            pltpu.SemaphoreType.DMA,
            pltpu.SemaphoreType.DMA,
        ],
    )
    def sc_copy(table_hbm, out_hbm, buf0, buf1, si0, si1, so0, so1):
        wid = lax.axis_index("s") * 2 + lax.axis_index("c")
        base = pl.multiple_of(wid * per_w, 8)
        bufs = (buf0, buf1)
        sin = (si0, si1)
        sout = (so0, so1)

        def start_in(c, b):
            return pltpu.async_copy(
                table_hbm.at[pl.ds(elem_start + base + offs[c], sizes[c])],
                bufs[b].at[pl.ds(0, sizes[c])],
                sin[b],
            )

        def start_out(c, b):
            return pltpu.async_copy(
                bufs[b].at[pl.ds(0, sizes[c])],
                out_hbm.at[pl.ds(base + offs[c], sizes[c])],
                sout[b],
            )

        in_cp = [None, None]
        out_cp = [None, None]
        in_cp[0] = start_in(0, 0)
        for c in range(n_chunks):
            b = c & 1
            nb = 1 - b
            in_cp[b].wait()
            if c + 1 < n_chunks:
                if out_cp[nb] is not None:
                    out_cp[nb].wait()
                in_cp[nb] = start_in(c + 1, nb)
            out_cp[b] = start_out(c, b)
        for b in range(2):
            if out_cp[b] is not None:
                out_cp[b].wait()

    return sc_copy


def kernel(x, rel_pos_emb):
    seq_len = x.shape[1]
    d = rel_pos_emb.shape[1]
    n_rows = 2 * seq_len - 1
    row_start = _MAX_LEN - seq_len + 1

    flat = rel_pos_emb.reshape(-1)
    out = _make_sc_copy(n_rows * d, row_start * d)(flat)
    return out.reshape(n_rows, 1, d)


# speedup vs baseline: 5.4112x; 2.0969x over previous
import functools

import jax
import jax.numpy as jnp
from jax import lax
from jax.experimental import pallas as pl
from jax.experimental.pallas import tpu as pltpu
from jax.experimental.pallas import tpu_sc as plsc

_MAX_LEN = 8000
_NUM_WORKERS = 32
_CHUNK_ROWS = 72


def _make_sc_lookup(n_rows: int, d: int, row_start: int):
    rows_per_w = (n_rows + _NUM_WORKERS - 1) // _NUM_WORKERS
    assert rows_per_w % 8 == 0
    aligned_start = row_start - row_start % 8
    shift = row_start % 8
    n_chunks = (rows_per_w + shift + _CHUNK_ROWS - 1) // _CHUNK_ROWS

    mesh = plsc.VectorSubcoreMesh(core_axis_name="c", subcore_axis_name="s")

    @functools.partial(
        pl.kernel,
        out_type=jax.ShapeDtypeStruct((n_rows * d,), jnp.float32),
        mesh=mesh,
        scratch_types=[
            pltpu.VMEM((_CHUNK_ROWS, d), jnp.float32),
            pltpu.SemaphoreType.DMA,
            pltpu.SemaphoreType.DMA,
        ],
    )
    def sc_lookup(table_hbm, out_hbm, buf, sem_in, sem_out):
        wid = lax.axis_index("s") * 2 + lax.axis_index("c")
        base = wid * rows_per_w

        for c in range(n_chunks):
            in_cp = pltpu.make_async_copy(
                table_hbm.at[
                    pl.ds(aligned_start + base + c * _CHUNK_ROWS, _CHUNK_ROWS), :
                ],
                buf,
                sem_in,
            )
            in_cp.start()
            in_cp.wait()

            lo = max(0, c * _CHUNK_ROWS - shift)
            hi = min(rows_per_w - 1, (c + 1) * _CHUNK_ROWS - 1 - shift)
            if lo > hi:
                continue
            hi_dyn = jnp.minimum(base + hi, n_rows - 1) - base

            def fire(i, _, c=c):
                ell = i + shift - c * _CHUNK_ROWS
                off = pl.multiple_of((base + i) * d, 8)
                pltpu.make_async_copy(
                    buf.at[ell], out_hbm.at[pl.ds(off, d)], sem_out
                ).start()
                return 0

            def drain(i, _):
                pltpu.make_async_copy(
                    buf.at[0], out_hbm.at[pl.ds(0, d)], sem_out
                ).wait()
                return 0

            lax.fori_loop(lo, hi_dyn + 1, fire, 0)
            lax.fori_loop(lo, hi_dyn + 1, drain, 0)

    return sc_lookup


def kernel(x, rel_pos_emb):
    seq_len = x.shape[1]
    d = rel_pos_emb.shape[1]
    n_rows = 2 * seq_len - 1
    row_start = _MAX_LEN - seq_len + 1

    out = _make_sc_lookup(n_rows, d, row_start)(rel_pos_emb)
    return out.reshape(n_rows, 1, d)


# speedup vs baseline: 5.5503x; 1.0257x over previous
import functools

import jax
import jax.numpy as jnp
from jax import lax
from jax.experimental import pallas as pl
from jax.experimental.pallas import tpu as pltpu
from jax.experimental.pallas import tpu_sc as plsc

_MAX_LEN = 8000
_NUM_WORKERS = 32
_CHUNK_ROWS = 56


def _make_sc_lookup(n_rows: int, d: int, row_start: int):
    rows_per_w = (n_rows + _NUM_WORKERS - 1) // _NUM_WORKERS
    shift = row_start % 8
    assert rows_per_w % 8 == 0 and 0 < shift
    n_chunks = (rows_per_w + shift + _CHUNK_ROWS - 1) // _CHUNK_ROWS

    spans = []
    for c in range(n_chunks):
        j0 = max(0, c * _CHUNK_ROWS - shift)
        j1 = min(rows_per_w - 1, (c + 1) * _CHUNK_ROWS - 1 - shift)
        spans.append((j0 - (c * _CHUNK_ROWS - shift), j0, j1 - j0 + 1))
    tail_drop = _NUM_WORKERS * rows_per_w - n_rows

    mesh = plsc.VectorSubcoreMesh(core_axis_name="c", subcore_axis_name="s")

    @functools.partial(
        pl.kernel,
        out_type=jax.ShapeDtypeStruct((n_rows, 1, d), jnp.float32),
        mesh=mesh,
        scratch_types=[
            pltpu.VMEM((_CHUNK_ROWS, 1, d), jnp.float32),
            pltpu.VMEM((_CHUNK_ROWS, 1, d), jnp.float32),
            pltpu.SemaphoreType.DMA,
            pltpu.SemaphoreType.DMA,
            pltpu.SemaphoreType.DMA,
            pltpu.SemaphoreType.DMA,
        ],
    )
    def sc_lookup(table_hbm, out_hbm, buf_a, buf_b, si_a, si_b, so_a, so_b):
        wid = lax.axis_index("s") * 2 + lax.axis_index("c")
        base = wid * rows_per_w
        a0 = (row_start - shift) + base
        bufs = (buf_a, buf_b)
        sin = (si_a, si_b)
        sout = (so_a, so_b)
        last = _NUM_WORKERS - 1

        def load(c):
            b = c % 2
            cp = pltpu.make_async_copy(
                table_hbm.at[
                    pl.ds(pl.multiple_of(a0 + c * _CHUNK_ROWS, 8), _CHUNK_ROWS), :
                ],
                bufs[b].at[:, 0, :],
                sin[b],
            )
            cp.start()
            return cp

        def store(c):
            b = c % 2
            ell, j0, k = spans[c]

            def fire(kk):
                pltpu.make_async_copy(
                    bufs[b].at[pl.ds(ell, kk), :, :],
                    out_hbm.at[pl.ds(base + j0, kk), :, :],
                    sout[b],
                ).start()

            if c == n_chunks - 1 and tail_drop:
                pl.when(wid < last)(lambda: fire(k))
                pl.when(wid == last)(lambda: fire(k - tail_drop))
            else:
                fire(k)

        def store_wait(c):
            b = c % 2
            ell, j0, k = spans[c]

            def drain(kk):
                pltpu.make_async_copy(
                    bufs[b].at[pl.ds(0, kk), :, :],
                    out_hbm.at[pl.ds(0, kk), :, :],
                    sout[b],
                ).wait()

            if c == n_chunks - 1 and tail_drop:
                pl.when(wid < last)(lambda: drain(k))
                pl.when(wid == last)(lambda: drain(k - tail_drop))
            else:
                drain(k)

        loads = {0: load(0)}
        for c in range(n_chunks):
            loads[c].wait()
            if c + 1 < n_chunks:
                if c >= 1:
                    store_wait(c - 1)
                loads[c + 1] = load(c + 1)
            store(c)
        if n_chunks >= 2:
            store_wait(n_chunks - 2)
        store_wait(n_chunks - 1)

    return sc_lookup


def kernel(x, rel_pos_emb):
    seq_len = x.shape[1]
    d = rel_pos_emb.shape[1]
    n_rows = 2 * seq_len - 1
    row_start = _MAX_LEN - seq_len + 1

    return _make_sc_lookup(n_rows, d, row_start)(rel_pos_emb)


# speedup vs baseline: 5.7402x; 1.0342x over previous
import functools

import jax
import jax.numpy as jnp
from jax import lax
from jax.experimental import pallas as pl
from jax.experimental.pallas import tpu as pltpu
from jax.experimental.pallas import tpu_sc as plsc

_MAX_LEN = 8000
_NUM_WORKERS = 32
_CHUNK_ROWS = 40
_N_BUFS = 3


def _make_sc_lookup(n_rows: int, d: int, row_start: int):
    rows_per_w = (n_rows + _NUM_WORKERS - 1) // _NUM_WORKERS
    shift = row_start % 8
    assert rows_per_w % 8 == 0 and 0 < shift
    n_chunks = (rows_per_w + shift + _CHUNK_ROWS - 1) // _CHUNK_ROWS

    spans = []
    for c in range(n_chunks):
        j0 = max(0, c * _CHUNK_ROWS - shift)
        j1 = min(rows_per_w - 1, (c + 1) * _CHUNK_ROWS - 1 - shift)
        spans.append((j0 - (c * _CHUNK_ROWS - shift), j0, j1 - j0 + 1))
    tail_drop = _NUM_WORKERS * rows_per_w - n_rows

    mesh = plsc.VectorSubcoreMesh(core_axis_name="c", subcore_axis_name="s")

    @functools.partial(
        pl.kernel,
        out_type=jax.ShapeDtypeStruct((n_rows, 1, d), jnp.float32),
        mesh=mesh,
        scratch_types=(
            [pltpu.VMEM((_CHUNK_ROWS, 1, d), jnp.float32)] * _N_BUFS
            + [pltpu.SemaphoreType.DMA] * (2 * _N_BUFS)
        ),
    )
    def sc_lookup(table_hbm, out_hbm, *scratch):
        bufs = scratch[:_N_BUFS]
        sin = scratch[_N_BUFS : 2 * _N_BUFS]
        sout = scratch[2 * _N_BUFS :]
        wid = lax.axis_index("s") * 2 + lax.axis_index("c")
        base = wid * rows_per_w
        a0 = (row_start - shift) + base
        last = _NUM_WORKERS - 1

        def load(c):
            b = c % _N_BUFS
            cp = pltpu.make_async_copy(
                table_hbm.at[
                    pl.ds(pl.multiple_of(a0 + c * _CHUNK_ROWS, 8), _CHUNK_ROWS), :
                ],
                bufs[b].at[:, 0, :],
                sin[b],
            )
            cp.start()
            return cp

        def store(c):
            b = c % _N_BUFS
            ell, j0, k = spans[c]

            def fire(kk):
                pltpu.make_async_copy(
                    bufs[b].at[pl.ds(ell, kk), :, :],
                    out_hbm.at[pl.ds(base + j0, kk), :, :],
                    sout[b],
                ).start()

            if c == n_chunks - 1 and tail_drop:
                pl.when(wid < last)(lambda: fire(k))
                pl.when(wid == last)(lambda: fire(k - tail_drop))
            else:
                fire(k)

        def store_wait(c):
            b = c % _N_BUFS
            ell, j0, k = spans[c]

            def drain(kk):
                pltpu.make_async_copy(
                    bufs[b].at[pl.ds(0, kk), :, :],
                    out_hbm.at[pl.ds(0, kk), :, :],
                    sout[b],
                ).wait()

            if c == n_chunks - 1 and tail_drop:
                pl.when(wid < last)(lambda: drain(k))
                pl.when(wid == last)(lambda: drain(k - tail_drop))
            else:
                drain(k)

        loads = {}
        for c in range(min(_N_BUFS - 1, n_chunks)):
            loads[c] = load(c)
        for c in range(n_chunks):
            loads[c].wait()
            nxt = c + _N_BUFS - 1
            if nxt < n_chunks:
                if c >= 1:
                    store_wait(c - 1)
                loads[nxt] = load(nxt)
            store(c)
        for c in range(max(0, n_chunks - _N_BUFS), n_chunks):
            store_wait(c)

    return sc_lookup


def kernel(x, rel_pos_emb):
    seq_len = x.shape[1]
    d = rel_pos_emb.shape[1]
    n_rows = 2 * seq_len - 1
    row_start = _MAX_LEN - seq_len + 1

    return _make_sc_lookup(n_rows, d, row_start)(rel_pos_emb)
